# Initial kernel scaffold; baseline (speedup 1.0000x reference)
import jax
import jax.numpy as jnp
from jax.experimental import pallas as pl


def kernel(x, edge_index, W1, b1, W2, b2, Wlin, blin):
    raise NotImplementedError("write your pallas kernel here")


# speedup vs baseline: 20.2629x; 20.2629x over previous
import jax
import jax.numpy as jnp
from jax import lax
from jax.experimental import pallas as pl
from jax.experimental.pallas import tpu as pltpu
from jax.experimental.pallas import tpu_sc as plsc

N = 10000
D = 128
E = 320000

NC = 2
NS = 16
NW = NC * NS

CH = 125
ROWS_W = E // NW // CH
N_ACC = 10240
ROWS_T = N_ACC // NS
ZCH = 64

_f32 = jnp.float32


def _deg_body(dst2d, ones_hbm, zeros_hbm, dpart, dacc, dbuf, ones):
    c = lax.axis_index("c")
    s = lax.axis_index("s")
    wid = s * NC + c

    pltpu.sync_copy(ones_hbm, ones)
    pltpu.sync_copy(zeros_hbm.at[pl.ds(s * ROWS_T, ROWS_T)],
                    dacc.at[pl.ds(s * ROWS_T, ROWS_T)])
    plsc.subcore_barrier()

    pltpu.sync_copy(dst2d.at[pl.ds(wid * ROWS_W, ROWS_W)], dbuf)

    def _chunk(j, _):
        pltpu.sync_copy(ones, dacc.at[dbuf.at[j]], add=True)
        return _
    lax.fori_loop(0, ROWS_W, _chunk, None)
    plsc.subcore_barrier()

    pltpu.sync_copy(dacc.at[pl.ds(s * ROWS_T, ROWS_T)],
                    dpart.at[c, pl.ds(s * ROWS_T, ROWS_T)])


def _deg_call(dst2d, ones_hbm, zeros_hbm):
    return pl.kernel(
        _deg_body,
        out_type=jax.ShapeDtypeStruct((NC, N_ACC, D), _f32),
        mesh=plsc.VectorSubcoreMesh(core_axis_name="c", subcore_axis_name="s",
                                    num_cores=NC, num_subcores=NS),
        scratch_types=[
            pltpu.VMEM_SHARED((N_ACC, D), _f32),
            pltpu.VMEM((ROWS_W, CH), jnp.int32),
            pltpu.VMEM((CH, D), _f32),
        ],
    )(dst2d, ones_hbm, zeros_hbm)


def _agg_body(g, src2d, dst2d, zeros_hbm, part, acc, sbuf, dbuf, rows, sem):
    c = lax.axis_index("c")
    s = lax.axis_index("s")
    wid = s * NC + c

    pltpu.sync_copy(zeros_hbm.at[pl.ds(s * ROWS_T, ROWS_T)],
                    acc.at[pl.ds(s * ROWS_T, ROWS_T)])
    plsc.subcore_barrier()

    pltpu.sync_copy(src2d.at[pl.ds(wid * ROWS_W, ROWS_W)], sbuf)
    pltpu.sync_copy(dst2d.at[pl.ds(wid * ROWS_W, ROWS_W)], dbuf)

    def _chunk(j, _):
        pltpu.async_copy(g.at[sbuf.at[j]], rows, sem).wait()
        pltpu.sync_copy(rows, acc.at[dbuf.at[j]], add=True)
        return _
    lax.fori_loop(0, ROWS_W, _chunk, None)
    plsc.subcore_barrier()

    pltpu.sync_copy(acc.at[pl.ds(s * ROWS_T, ROWS_T)],
                    part.at[c, pl.ds(s * ROWS_T, ROWS_T)])


def _agg_call(g, src2d, dst2d, zeros_hbm):
    return pl.kernel(
        _agg_body,
        out_type=jax.ShapeDtypeStruct((NC, N_ACC, D), _f32),
        mesh=plsc.VectorSubcoreMesh(core_axis_name="c", subcore_axis_name="s",
                                    num_cores=NC, num_subcores=NS),
        scratch_types=[
            pltpu.VMEM_SHARED((N_ACC, D), _f32),
            pltpu.VMEM((ROWS_W, CH), jnp.int32),
            pltpu.VMEM((ROWS_W, CH), jnp.int32),
            pltpu.VMEM((CH, D), _f32),
            pltpu.SemaphoreType.DMA,
        ],
    )(g, src2d, dst2d, zeros_hbm)


_R = 2000


def _dis(degp):
    return lax.rsqrt(1.0 + degp[0, :, 0] + degp[1, :, 0])[:, None]


def _gscale_body(degp_ref, x_ref, w_ref, g_ref):
    dis = _dis(degp_ref[...])
    h = jnp.dot(x_ref[...], w_ref[...], preferred_element_type=_f32)
    g_ref[...] = dis * h


def _gscale_call(degp, x, w):
    return pl.pallas_call(
        _gscale_body,
        grid=(N // _R,),
        in_specs=[
            pl.BlockSpec((NC, _R, D), lambda i: (0, i, 0)),
            pl.BlockSpec((_R, D), lambda i: (i, 0)),
            pl.BlockSpec((D, D), lambda i: (0, 0)),
        ],
        out_specs=pl.BlockSpec((_R, D), lambda i: (i, 0)),
        out_shape=jax.ShapeDtypeStruct((N, D), _f32),
    )(degp, x, w)


def _layer_body(degp_ref, p_ref, g_ref, b_ref, w_ref, g2_ref):
    dis = _dis(degp_ref[...])
    p = p_ref[...]
    agg = p[0] + p[1] + g_ref[...]
    out1 = jnp.maximum(dis * agg + b_ref[...], 0.0)
    g2_ref[...] = dis * jnp.dot(out1, w_ref[...], preferred_element_type=_f32)


def _layer_call(degp, part, g, b2d, w):
    return pl.pallas_call(
        _layer_body,
        grid=(N // _R,),
        in_specs=[
            pl.BlockSpec((NC, _R, D), lambda i: (0, i, 0)),
            pl.BlockSpec((NC, _R, D), lambda i: (0, i, 0)),
            pl.BlockSpec((_R, D), lambda i: (i, 0)),
            pl.BlockSpec((1, D), lambda i: (0, 0)),
            pl.BlockSpec((D, D), lambda i: (0, 0)),
        ],
        out_specs=pl.BlockSpec((_R, D), lambda i: (i, 0)),
        out_shape=jax.ShapeDtypeStruct((N, D), _f32),
    )(degp, part, g, b2d, w)


def _final_body(degp_ref, p_ref, g_ref, b_ref, wl_ref, bl_ref, res_ref, acc_ref):
    i = pl.program_id(0)
    dis = _dis(degp_ref[...])
    p = p_ref[...]
    agg = p[0] + p[1] + g_ref[...]
    out2 = jnp.maximum(dis * agg + b_ref[...], 0.0)
    psum = jnp.sum(out2, axis=0, keepdims=True)

    @pl.when(i == 0)
    def _():
        acc_ref[...] = jnp.zeros_like(acc_ref)

    acc_ref[...] += psum

    @pl.when(i == pl.num_programs(0) - 1)
    def _():
        m = acc_ref[...] * (1.0 / N)
        y = jnp.dot(m, wl_ref[...], preferred_element_type=_f32) + bl_ref[...]
        res_ref[...] = jax.nn.sigmoid(y)


def _final_call(degp, part, g, b2d, wl, bl2d):
    return pl.pallas_call(
        _final_body,
        grid=(N // _R,),
        in_specs=[
            pl.BlockSpec((NC, _R, D), lambda i: (0, i, 0)),
            pl.BlockSpec((NC, _R, D), lambda i: (0, i, 0)),
            pl.BlockSpec((_R, D), lambda i: (i, 0)),
            pl.BlockSpec((1, D), lambda i: (0, 0)),
            pl.BlockSpec((D, 2), lambda i: (0, 0)),
            pl.BlockSpec((1, 2), lambda i: (0, 0)),
        ],
        out_specs=pl.BlockSpec((1, 2), lambda i: (0, 0)),
        out_shape=jax.ShapeDtypeStruct((1, 2), _f32),
        scratch_shapes=[pltpu.VMEM((1, D), _f32)],
    )(degp, part, g, b2d, wl, bl2d)


def kernel(x, edge_index, W1, b1, W2, b2, Wlin, blin):
    ei = edge_index.astype(jnp.int32)
    src2d = ei[0].reshape(E // CH, CH)
    dst2d = ei[1].reshape(E // CH, CH)
    onesD = jnp.ones((CH, D), _f32)
    zerosD = jnp.zeros((N_ACC, D), _f32)

    degp = _deg_call(dst2d, onesD, zerosD)
    g1 = _gscale_call(degp, x, W1)
    part1 = _agg_call(g1, src2d, dst2d, zerosD)
    g2 = _layer_call(degp, part1, g1, b1[None, :], W2)
    part2 = _agg_call(g2, src2d, dst2d, zerosD)
    res = _final_call(degp, part2, g2, b2[None, :], Wlin, blin[None, :])
    return res[0]


# speedup vs baseline: 23.1617x; 1.1431x over previous
import jax
import jax.numpy as jnp
from jax import lax
from jax.experimental import pallas as pl
from jax.experimental.pallas import tpu as pltpu
from jax.experimental.pallas import tpu_sc as plsc

N = 10000
D = 128
E = 320000

NC = 2
NS = 16
NW = NC * NS

CH = 125
ROWS_W = E // NW // CH
N_ACC = 10240
ROWS_T = N_ACC // NS
ZCH = 64

_f32 = jnp.float32


def _deg_body(dst2d, ones_hbm, zeros_hbm, dpart, dacc, dbuf, ones):
    c = lax.axis_index("c")
    s = lax.axis_index("s")
    wid = s * NC + c

    pltpu.sync_copy(ones_hbm, ones)
    pltpu.sync_copy(zeros_hbm.at[pl.ds(s * ROWS_T, ROWS_T)],
                    dacc.at[pl.ds(s * ROWS_T, ROWS_T)])
    plsc.subcore_barrier()

    pltpu.sync_copy(dst2d.at[pl.ds(wid * ROWS_W, ROWS_W)], dbuf)

    def _chunk(j, _):
        pltpu.sync_copy(ones, dacc.at[dbuf.at[j]], add=True)
        return _
    lax.fori_loop(0, ROWS_W, _chunk, None)
    plsc.subcore_barrier()

    pltpu.sync_copy(dacc.at[pl.ds(s * ROWS_T, ROWS_T)],
                    dpart.at[c, pl.ds(s * ROWS_T, ROWS_T)])


def _deg_call(dst2d, ones_hbm, zeros_hbm):
    return pl.kernel(
        _deg_body,
        out_type=jax.ShapeDtypeStruct((NC, N_ACC, D), _f32),
        mesh=plsc.VectorSubcoreMesh(core_axis_name="c", subcore_axis_name="s",
                                    num_cores=NC, num_subcores=NS),
        scratch_types=[
            pltpu.VMEM_SHARED((N_ACC, D), _f32),
            pltpu.VMEM((ROWS_W, CH), jnp.int32),
            pltpu.VMEM((CH, D), _f32),
        ],
    )(dst2d, ones_hbm, zeros_hbm)


HW = ROWS_W // 2


def _agg_body(g, src2d, dst2d, zeros_hbm, part, acc,
              sbuf, dbuf, rows0, rows1, gs0, gs1, ss0, ss1):
    c = lax.axis_index("c")
    s = lax.axis_index("s")
    wid = s * NC + c

    pltpu.sync_copy(zeros_hbm.at[pl.ds(s * ROWS_T, ROWS_T)],
                    acc.at[pl.ds(s * ROWS_T, ROWS_T)])
    plsc.subcore_barrier()

    base = wid * ROWS_W
    for h in range(2):
        pltpu.sync_copy(src2d.at[pl.ds(base + h * HW, HW)], sbuf)
        pltpu.sync_copy(dst2d.at[pl.ds(base + h * HW, HW)], dbuf)
        pltpu.async_copy(g.at[sbuf.at[0]], rows0, gs0)
        pltpu.async_copy(g.at[sbuf.at[1]], rows1, gs1)

        def _pair(t, _):
            j0 = 2 * t
            j1 = j0 + 1
            pltpu.make_async_copy(g.at[sbuf.at[j0]], rows0, gs0).wait()
            pltpu.async_copy(rows0, acc.at[dbuf.at[j0]], ss0, add=True)
            pltpu.make_async_copy(g.at[sbuf.at[j1]], rows1, gs1).wait()
            pltpu.async_copy(rows1, acc.at[dbuf.at[j1]], ss1, add=True)

            @pl.when(t + 1 < HW // 2)
            def _():
                pltpu.make_async_copy(rows0, acc.at[dbuf.at[j0]], ss0).wait()
                pltpu.async_copy(g.at[sbuf.at[j0 + 2]], rows0, gs0)
                pltpu.make_async_copy(rows1, acc.at[dbuf.at[j1]], ss1).wait()
                pltpu.async_copy(g.at[sbuf.at[j1 + 2]], rows1, gs1)
            return _

        lax.fori_loop(0, HW // 2, _pair, None)
        pltpu.make_async_copy(rows0, acc.at[dbuf.at[HW - 2]], ss0).wait()
        pltpu.make_async_copy(rows1, acc.at[dbuf.at[HW - 1]], ss1).wait()
    plsc.subcore_barrier()

    pltpu.sync_copy(acc.at[pl.ds(s * ROWS_T, ROWS_T)],
                    part.at[c, pl.ds(s * ROWS_T, ROWS_T)])


def _agg_call(g, src2d, dst2d, zeros_hbm):
    return pl.kernel(
        _agg_body,
        out_type=jax.ShapeDtypeStruct((NC, N_ACC, D), _f32),
        mesh=plsc.VectorSubcoreMesh(core_axis_name="c", subcore_axis_name="s",
                                    num_cores=NC, num_subcores=NS),
        scratch_types=[
            pltpu.VMEM_SHARED((N_ACC, D), _f32),
            pltpu.VMEM((HW, CH), jnp.int32),
            pltpu.VMEM((HW, CH), jnp.int32),
            pltpu.VMEM((CH, D), _f32),
            pltpu.VMEM((CH, D), _f32),
            pltpu.SemaphoreType.DMA,
            pltpu.SemaphoreType.DMA,
            pltpu.SemaphoreType.DMA,
            pltpu.SemaphoreType.DMA,
        ],
    )(g, src2d, dst2d, zeros_hbm)


_R = 2000


def _dis(degp):
    return lax.rsqrt(1.0 + degp[0, :, 0] + degp[1, :, 0])[:, None]


def _gscale_body(degp_ref, x_ref, w_ref, g_ref):
    dis = _dis(degp_ref[...])
    h = jnp.dot(x_ref[...], w_ref[...], preferred_element_type=_f32)
    g_ref[...] = dis * h


def _gscale_call(degp, x, w):
    return pl.pallas_call(
        _gscale_body,
        grid=(N // _R,),
        in_specs=[
            pl.BlockSpec((NC, _R, D), lambda i: (0, i, 0)),
            pl.BlockSpec((_R, D), lambda i: (i, 0)),
            pl.BlockSpec((D, D), lambda i: (0, 0)),
        ],
        out_specs=pl.BlockSpec((_R, D), lambda i: (i, 0)),
        out_shape=jax.ShapeDtypeStruct((N, D), _f32),
    )(degp, x, w)


def _layer_body(degp_ref, p_ref, g_ref, b_ref, w_ref, g2_ref):
    dis = _dis(degp_ref[...])
    p = p_ref[...]
    agg = p[0] + p[1] + g_ref[...]
    out1 = jnp.maximum(dis * agg + b_ref[...], 0.0)
    g2_ref[...] = dis * jnp.dot(out1, w_ref[...], preferred_element_type=_f32)


def _layer_call(degp, part, g, b2d, w):
    return pl.pallas_call(
        _layer_body,
        grid=(N // _R,),
        in_specs=[
            pl.BlockSpec((NC, _R, D), lambda i: (0, i, 0)),
            pl.BlockSpec((NC, _R, D), lambda i: (0, i, 0)),
            pl.BlockSpec((_R, D), lambda i: (i, 0)),
            pl.BlockSpec((1, D), lambda i: (0, 0)),
            pl.BlockSpec((D, D), lambda i: (0, 0)),
        ],
        out_specs=pl.BlockSpec((_R, D), lambda i: (i, 0)),
        out_shape=jax.ShapeDtypeStruct((N, D), _f32),
    )(degp, part, g, b2d, w)


def _final_body(degp_ref, p_ref, g_ref, b_ref, wl_ref, bl_ref, res_ref, acc_ref):
    i = pl.program_id(0)
    dis = _dis(degp_ref[...])
    p = p_ref[...]
    agg = p[0] + p[1] + g_ref[...]
    out2 = jnp.maximum(dis * agg + b_ref[...], 0.0)
    psum = jnp.sum(out2, axis=0, keepdims=True)

    @pl.when(i == 0)
    def _():
        acc_ref[...] = jnp.zeros_like(acc_ref)

    acc_ref[...] += psum

    @pl.when(i == pl.num_programs(0) - 1)
    def _():
        m = acc_ref[...] * (1.0 / N)
        y = jnp.dot(m, wl_ref[...], preferred_element_type=_f32) + bl_ref[...]
        res_ref[...] = jax.nn.sigmoid(y)


def _final_call(degp, part, g, b2d, wl, bl2d):
    return pl.pallas_call(
        _final_body,
        grid=(N // _R,),
        in_specs=[
            pl.BlockSpec((NC, _R, D), lambda i: (0, i, 0)),
            pl.BlockSpec((NC, _R, D), lambda i: (0, i, 0)),
            pl.BlockSpec((_R, D), lambda i: (i, 0)),
            pl.BlockSpec((1, D), lambda i: (0, 0)),
            pl.BlockSpec((D, 2), lambda i: (0, 0)),
            pl.BlockSpec((1, 2), lambda i: (0, 0)),
        ],
        out_specs=pl.BlockSpec((1, 2), lambda i: (0, 0)),
        out_shape=jax.ShapeDtypeStruct((1, 2), _f32),
        scratch_shapes=[pltpu.VMEM((1, D), _f32)],
    )(degp, part, g, b2d, wl, bl2d)


def kernel(x, edge_index, W1, b1, W2, b2, Wlin, blin):
    ei = edge_index.astype(jnp.int32)
    src2d = ei[0].reshape(E // CH, CH)
    dst2d = ei[1].reshape(E // CH, CH)
    onesD = jnp.ones((CH, D), _f32)
    zerosD = jnp.zeros((N_ACC, D), _f32)

    degp = _deg_call(dst2d, onesD, zerosD)
    g1 = _gscale_call(degp, x, W1)
    part1 = _agg_call(g1, src2d, dst2d, zerosD)
    g2 = _layer_call(degp, part1, g1, b1[None, :], W2)
    part2 = _agg_call(g2, src2d, dst2d, zerosD)
    res = _final_call(degp, part2, g2, b2[None, :], Wlin, blin[None, :])
    return res[0]


# speedup vs baseline: 25.7269x; 1.1108x over previous
import jax
import jax.numpy as jnp
from jax import lax
from jax.experimental import pallas as pl
from jax.experimental.pallas import tpu as pltpu
from jax.experimental.pallas import tpu_sc as plsc

N = 10000
D = 128
E = 320000

NC = 2
NS = 16
NW = NC * NS

CH = 50
ROWS_W = E // NW // CH
N_ACC = 10240
ROWS_T = N_ACC // NS
NBUF = 4
DRAIN = 8

_f32 = jnp.float32


def _deg_body(dst2d, ones_hbm, zeros_hbm, dpart, dacc, dbuf, ones, dsem):
    c = lax.axis_index("c")
    s = lax.axis_index("s")
    wid = s * NC + c

    pltpu.sync_copy(ones_hbm, ones)
    pltpu.sync_copy(zeros_hbm.at[pl.ds(s * ROWS_T, ROWS_T)],
                    dacc.at[pl.ds(s * ROWS_T, ROWS_T)])
    plsc.subcore_barrier()

    pltpu.sync_copy(dst2d.at[pl.ds(wid * ROWS_W, ROWS_W)], dbuf)

    def _group(t, _):
        for b in range(DRAIN):
            pltpu.async_copy(ones, dacc.at[dbuf.at[t * DRAIN + b]], dsem,
                             add=True)
        for b in range(DRAIN):
            pltpu.make_async_copy(ones, dacc.at[dbuf.at[t * DRAIN + b]],
                                  dsem).wait()
        return _
    lax.fori_loop(0, ROWS_W // DRAIN, _group, None)
    plsc.subcore_barrier()

    pltpu.sync_copy(dacc.at[pl.ds(s * ROWS_T, ROWS_T)],
                    dpart.at[c, pl.ds(s * ROWS_T, ROWS_T)])


def _deg_call(dst2d, ones_hbm, zeros_hbm):
    return pl.kernel(
        _deg_body,
        out_type=jax.ShapeDtypeStruct((NC, N_ACC, D), _f32),
        mesh=plsc.VectorSubcoreMesh(core_axis_name="c", subcore_axis_name="s",
                                    num_cores=NC, num_subcores=NS),
        scratch_types=[
            pltpu.VMEM_SHARED((N_ACC, D), _f32),
            pltpu.VMEM((ROWS_W, CH), jnp.int32),
            pltpu.VMEM((CH, D), _f32),
            pltpu.SemaphoreType.DMA,
        ],
    )(dst2d, ones_hbm, zeros_hbm)


def _agg_body(g, src2d, dst2d, zeros_hbm, part, acc,
              sbuf, dbuf, rows, gsems, ssems):
    c = lax.axis_index("c")
    s = lax.axis_index("s")
    wid = s * NC + c

    pltpu.sync_copy(zeros_hbm.at[pl.ds(s * ROWS_T, ROWS_T)],
                    acc.at[pl.ds(s * ROWS_T, ROWS_T)])
    plsc.subcore_barrier()

    base = wid * ROWS_W
    for off, HW in ((0, 56), (56, 48), (104, 48), (152, 48)):
        NGRP = HW // NBUF
        pltpu.sync_copy(src2d.at[pl.ds(base + off, HW)], sbuf.at[pl.ds(0, HW)])
        pltpu.sync_copy(dst2d.at[pl.ds(base + off, HW)], dbuf.at[pl.ds(0, HW)])

        for b in range(NBUF):
            pltpu.async_copy(g.at[sbuf.at[b]], rows[b], gsems[b])

        def _grp(t, _):
            j = t * NBUF
            for b in range(NBUF):
                pltpu.make_async_copy(g.at[sbuf.at[j + b]], rows[b],
                                      gsems[b]).wait()
                pltpu.async_copy(rows[b], acc.at[dbuf.at[j + b]], ssems[b],
                                 add=True)
            @pl.when(t + 1 < NGRP)
            def _():
                for b in range(NBUF):
                    pltpu.make_async_copy(rows[b], acc.at[dbuf.at[j + b]],
                                          ssems[b]).wait()
                    pltpu.async_copy(g.at[sbuf.at[j + NBUF + b]], rows[b],
                                     gsems[b])
            return _

        lax.fori_loop(0, NGRP, _grp, None)
        for b in range(NBUF):
            pltpu.make_async_copy(rows[b], acc.at[dbuf.at[HW - NBUF + b]],
                                  ssems[b]).wait()
    plsc.subcore_barrier()

    pltpu.sync_copy(acc.at[pl.ds(s * ROWS_T, ROWS_T)],
                    part.at[c, pl.ds(s * ROWS_T, ROWS_T)])


def _agg_call(g, src2d, dst2d, zeros_hbm):
    return pl.kernel(
        _agg_body,
        out_type=jax.ShapeDtypeStruct((NC, N_ACC, D), _f32),
        mesh=plsc.VectorSubcoreMesh(core_axis_name="c", subcore_axis_name="s",
                                    num_cores=NC, num_subcores=NS),
        scratch_types=[
            pltpu.VMEM_SHARED((N_ACC, D), _f32),
            pltpu.VMEM((56, CH), jnp.int32),
            pltpu.VMEM((56, CH), jnp.int32),
            [pltpu.VMEM((CH, D), _f32)] * NBUF,
            [pltpu.SemaphoreType.DMA] * NBUF,
            [pltpu.SemaphoreType.DMA] * NBUF,
        ],
    )(g, src2d, dst2d, zeros_hbm)


_R = 2000


def _dis(degp):
    return lax.rsqrt(1.0 + degp[0, :, 0] + degp[1, :, 0])[:, None]


def _gscale_body(degp_ref, x_ref, w_ref, g_ref):
    dis = _dis(degp_ref[...])
    h = jnp.dot(x_ref[...], w_ref[...], preferred_element_type=_f32)
    g_ref[...] = dis * h


def _gscale_call(degp, x, w):
    return pl.pallas_call(
        _gscale_body,
        grid=(N // _R,),
        in_specs=[
            pl.BlockSpec((NC, _R, D), lambda i: (0, i, 0)),
            pl.BlockSpec((_R, D), lambda i: (i, 0)),
            pl.BlockSpec((D, D), lambda i: (0, 0)),
        ],
        out_specs=pl.BlockSpec((_R, D), lambda i: (i, 0)),
        out_shape=jax.ShapeDtypeStruct((N, D), _f32),
    )(degp, x, w)


def _layer_body(degp_ref, p_ref, g_ref, b_ref, w_ref, g2_ref):
    dis = _dis(degp_ref[...])
    p = p_ref[...]
    agg = p[0] + p[1] + g_ref[...]
    out1 = jnp.maximum(dis * agg + b_ref[...], 0.0)
    g2_ref[...] = dis * jnp.dot(out1, w_ref[...], preferred_element_type=_f32)


def _layer_call(degp, part, g, b2d, w):
    return pl.pallas_call(
        _layer_body,
        grid=(N // _R,),
        in_specs=[
            pl.BlockSpec((NC, _R, D), lambda i: (0, i, 0)),
            pl.BlockSpec((NC, _R, D), lambda i: (0, i, 0)),
            pl.BlockSpec((_R, D), lambda i: (i, 0)),
            pl.BlockSpec((1, D), lambda i: (0, 0)),
            pl.BlockSpec((D, D), lambda i: (0, 0)),
        ],
        out_specs=pl.BlockSpec((_R, D), lambda i: (i, 0)),
        out_shape=jax.ShapeDtypeStruct((N, D), _f32),
    )(degp, part, g, b2d, w)


def _final_body(degp_ref, p_ref, g_ref, b_ref, wl_ref, bl_ref, res_ref, acc_ref):
    i = pl.program_id(0)
    dis = _dis(degp_ref[...])
    p = p_ref[...]
    agg = p[0] + p[1] + g_ref[...]
    out2 = jnp.maximum(dis * agg + b_ref[...], 0.0)
    psum = jnp.sum(out2, axis=0, keepdims=True)

    @pl.when(i == 0)
    def _():
        acc_ref[...] = jnp.zeros_like(acc_ref)

    acc_ref[...] += psum

    @pl.when(i == pl.num_programs(0) - 1)
    def _():
        m = acc_ref[...] * (1.0 / N)
        y = jnp.dot(m, wl_ref[...], preferred_element_type=_f32) + bl_ref[...]
        res_ref[...] = jax.nn.sigmoid(y)


def _final_call(degp, part, g, b2d, wl, bl2d):
    return pl.pallas_call(
        _final_body,
        grid=(N // _R,),
        in_specs=[
            pl.BlockSpec((NC, _R, D), lambda i: (0, i, 0)),
            pl.BlockSpec((NC, _R, D), lambda i: (0, i, 0)),
            pl.BlockSpec((_R, D), lambda i: (i, 0)),
            pl.BlockSpec((1, D), lambda i: (0, 0)),
            pl.BlockSpec((D, 2), lambda i: (0, 0)),
            pl.BlockSpec((1, 2), lambda i: (0, 0)),
        ],
        out_specs=pl.BlockSpec((1, 2), lambda i: (0, 0)),
        out_shape=jax.ShapeDtypeStruct((1, 2), _f32),
        scratch_shapes=[pltpu.VMEM((1, D), _f32)],
    )(degp, part, g, b2d, wl, bl2d)


def kernel(x, edge_index, W1, b1, W2, b2, Wlin, blin):
    ei = edge_index.astype(jnp.int32)
    src2d = ei[0].reshape(E // CH, CH)
    dst2d = ei[1].reshape(E // CH, CH)
    onesD = jnp.ones((CH, D), _f32)
    zerosD = jnp.zeros((N_ACC, D), _f32)

    degp = _deg_call(dst2d, onesD, zerosD)
    g1 = _gscale_call(degp, x, W1)
    part1 = _agg_call(g1, src2d, dst2d, zerosD)
    g2 = _layer_call(degp, part1, g1, b1[None, :], W2)
    part2 = _agg_call(g2, src2d, dst2d, zerosD)
    res = _final_call(degp, part2, g2, b2[None, :], Wlin, blin[None, :])
    return res[0]


# speedup vs baseline: 25.7556x; 1.0011x over previous
import jax
import jax.numpy as jnp
from jax import lax
from jax.experimental import pallas as pl
from jax.experimental.pallas import tpu as pltpu
from jax.experimental.pallas import tpu_sc as plsc

N = 10000
D = 128
E = 320000

NC = 2
NS = 16
NW = NC * NS

CH = 50
ROWS_W = E // NW // CH
N_ACC = 10240
ROWS_T = N_ACC // NS
NBUF = 4
DRAIN = 8

_f32 = jnp.float32


def _deg_body(dst2d, ones_hbm, zeros_hbm, dpart, dacc, dbuf, ones, dsem):
    c = lax.axis_index("c")
    s = lax.axis_index("s")
    wid = s * NC + c

    pltpu.sync_copy(ones_hbm, ones)
    pltpu.sync_copy(zeros_hbm.at[pl.ds(s * ROWS_T, ROWS_T)],
                    dacc.at[pl.ds(s * ROWS_T, ROWS_T)])
    plsc.subcore_barrier()

    pltpu.sync_copy(dst2d.at[pl.ds(wid * ROWS_W, ROWS_W)], dbuf)

    def _group(t, _):
        for b in range(DRAIN):
            pltpu.async_copy(ones, dacc.at[dbuf.at[t * DRAIN + b]], dsem,
                             add=True)
        for b in range(DRAIN):
            pltpu.make_async_copy(ones, dacc.at[dbuf.at[t * DRAIN + b]],
                                  dsem).wait()
        return _
    lax.fori_loop(0, ROWS_W // DRAIN, _group, None)
    plsc.subcore_barrier()

    pltpu.sync_copy(dacc.at[pl.ds(s * ROWS_T, ROWS_T)],
                    dpart.at[c, pl.ds(s * ROWS_T, ROWS_T)])


def _deg_call(dst2d, ones_hbm, zeros_hbm):
    return pl.kernel(
        _deg_body,
        out_type=jax.ShapeDtypeStruct((NC, N_ACC, D), _f32),
        mesh=plsc.VectorSubcoreMesh(core_axis_name="c", subcore_axis_name="s",
                                    num_cores=NC, num_subcores=NS),
        scratch_types=[
            pltpu.VMEM_SHARED((N_ACC, D), _f32),
            pltpu.VMEM((ROWS_W, CH), jnp.int32),
            pltpu.VMEM((CH, D), _f32),
            pltpu.SemaphoreType.DMA,
        ],
    )(dst2d, ones_hbm, zeros_hbm)


def _agg_body(g, src2d, dst2d, zeros_hbm, part, acc,
              sbuf, dbuf, rows, gsems, ssems):
    c = lax.axis_index("c")
    s = lax.axis_index("s")
    wid = s * NC + c

    pltpu.sync_copy(zeros_hbm.at[pl.ds(s * ROWS_T, ROWS_T)],
                    acc.at[pl.ds(s * ROWS_T, ROWS_T)])
    plsc.subcore_barrier()

    base = wid * ROWS_W
    for off, HW in ((0, 56), (56, 48), (104, 48), (152, 48)):
        NGRP = HW // NBUF
        pltpu.sync_copy(src2d.at[pl.ds(base + off, HW)], sbuf.at[pl.ds(0, HW)])
        pltpu.sync_copy(dst2d.at[pl.ds(base + off, HW)], dbuf.at[pl.ds(0, HW)])

        for b in range(NBUF):
            pltpu.async_copy(g.at[sbuf.at[b]], rows[b], gsems[b])

        def _grp(t, _):
            j = t * NBUF
            for b in range(NBUF):
                pltpu.make_async_copy(g.at[sbuf.at[j + b]], rows[b],
                                      gsems[b]).wait()
                pltpu.async_copy(rows[b], acc.at[dbuf.at[j + b]], ssems[b],
                                 add=True)
            @pl.when(t + 1 < NGRP)
            def _():
                for b in range(NBUF):
                    pltpu.make_async_copy(rows[b], acc.at[dbuf.at[j + b]],
                                          ssems[b]).wait()
                    pltpu.async_copy(g.at[sbuf.at[j + NBUF + b]], rows[b],
                                     gsems[b])
            return _

        lax.fori_loop(0, NGRP, _grp, None)
        for b in range(NBUF):
            pltpu.make_async_copy(rows[b], acc.at[dbuf.at[HW - NBUF + b]],
                                  ssems[b]).wait()
    plsc.subcore_barrier()

    pltpu.sync_copy(acc.at[pl.ds(s * ROWS_T, ROWS_T)],
                    part.at[c, pl.ds(s * ROWS_T, ROWS_T)])


def _agg_call(g, src2d, dst2d, zeros_hbm):
    return pl.kernel(
        _agg_body,
        out_type=jax.ShapeDtypeStruct((NC, N_ACC, D), _f32),
        mesh=plsc.VectorSubcoreMesh(core_axis_name="c", subcore_axis_name="s",
                                    num_cores=NC, num_subcores=NS),
        scratch_types=[
            pltpu.VMEM_SHARED((N_ACC, D), _f32),
            pltpu.VMEM((56, CH), jnp.int32),
            pltpu.VMEM((56, CH), jnp.int32),
            [pltpu.VMEM((CH, D), _f32)] * NBUF,
            [pltpu.SemaphoreType.DMA] * NBUF,
            [pltpu.SemaphoreType.DMA] * NBUF,
        ],
    )(g, src2d, dst2d, zeros_hbm)


_R = 2000


def _dis(degp):
    return lax.rsqrt(1.0 + degp[0, :, 0] + degp[1, :, 0])[:, None]


def _mm_body(x_ref, w_ref, h_ref):
    h_ref[...] = jnp.dot(x_ref[...], w_ref[...], preferred_element_type=_f32)


def _mm_call(x, w):
    return pl.pallas_call(
        _mm_body,
        grid=(N // _R,),
        in_specs=[
            pl.BlockSpec((_R, D), lambda i: (i, 0)),
            pl.BlockSpec((D, D), lambda i: (0, 0)),
        ],
        out_specs=pl.BlockSpec((_R, D), lambda i: (i, 0)),
        out_shape=jax.ShapeDtypeStruct((N, D), _f32),
    )(x, w)


def _gscale_body(degp_ref, h_ref, g_ref, dis_ref):
    dis = _dis(degp_ref[...])
    g_ref[...] = dis * h_ref[...]
    dis_ref[...] = jnp.broadcast_to(dis, (dis.shape[0], 8))


def _gscale_call(degp, h):
    return pl.pallas_call(
        _gscale_body,
        grid=(N // _R,),
        in_specs=[
            pl.BlockSpec((NC, _R, D), lambda i: (0, i, 0)),
            pl.BlockSpec((_R, D), lambda i: (i, 0)),
        ],
        out_specs=[
            pl.BlockSpec((_R, D), lambda i: (i, 0)),
            pl.BlockSpec((_R, 8), lambda i: (i, 0)),
        ],
        out_shape=[
            jax.ShapeDtypeStruct((N, D), _f32),
            jax.ShapeDtypeStruct((N, 8), _f32),
        ],
    )(degp, h)


def _layer_body(dis_ref, p_ref, g_ref, b_ref, w_ref, g2_ref):
    dis = dis_ref[:, 0:1]
    p = p_ref[...]
    agg = p[0] + p[1] + g_ref[...]
    out1 = jnp.maximum(dis * agg + b_ref[...], 0.0)
    g2_ref[...] = dis * jnp.dot(out1, w_ref[...], preferred_element_type=_f32)


def _layer_call(dis8, part, g, b2d, w):
    return pl.pallas_call(
        _layer_body,
        grid=(N // _R,),
        in_specs=[
            pl.BlockSpec((_R, 8), lambda i: (i, 0)),
            pl.BlockSpec((NC, _R, D), lambda i: (0, i, 0)),
            pl.BlockSpec((_R, D), lambda i: (i, 0)),
            pl.BlockSpec((1, D), lambda i: (0, 0)),
            pl.BlockSpec((D, D), lambda i: (0, 0)),
        ],
        out_specs=pl.BlockSpec((_R, D), lambda i: (i, 0)),
        out_shape=jax.ShapeDtypeStruct((N, D), _f32),
    )(dis8, part, g, b2d, w)


def _final_body(dis_ref, p_ref, g_ref, b_ref, wl_ref, bl_ref, res_ref, acc_ref):
    i = pl.program_id(0)
    dis = dis_ref[:, 0:1]
    p = p_ref[...]
    agg = p[0] + p[1] + g_ref[...]
    out2 = jnp.maximum(dis * agg + b_ref[...], 0.0)
    psum = jnp.sum(out2, axis=0, keepdims=True)

    @pl.when(i == 0)
    def _():
        acc_ref[...] = jnp.zeros_like(acc_ref)

    acc_ref[...] += psum

    @pl.when(i == pl.num_programs(0) - 1)
    def _():
        m = acc_ref[...] * (1.0 / N)
        y = jnp.dot(m, wl_ref[...], preferred_element_type=_f32) + bl_ref[...]
        res_ref[...] = jax.nn.sigmoid(y)


def _final_call(dis8, part, g, b2d, wl, bl2d):
    return pl.pallas_call(
        _final_body,
        grid=(N // _R,),
        in_specs=[
            pl.BlockSpec((_R, 8), lambda i: (i, 0)),
            pl.BlockSpec((NC, _R, D), lambda i: (0, i, 0)),
            pl.BlockSpec((_R, D), lambda i: (i, 0)),
            pl.BlockSpec((1, D), lambda i: (0, 0)),
            pl.BlockSpec((D, 2), lambda i: (0, 0)),
            pl.BlockSpec((1, 2), lambda i: (0, 0)),
        ],
        out_specs=pl.BlockSpec((1, 2), lambda i: (0, 0)),
        out_shape=jax.ShapeDtypeStruct((1, 2), _f32),
        scratch_shapes=[pltpu.VMEM((1, D), _f32)],
    )(dis8, part, g, b2d, wl, bl2d)


def kernel(x, edge_index, W1, b1, W2, b2, Wlin, blin):
    ei = edge_index.astype(jnp.int32)
    src2d = ei[0].reshape(E // CH, CH)
    dst2d = ei[1].reshape(E // CH, CH)
    onesD = jnp.ones((CH, D), _f32)
    zerosD = jnp.zeros((N_ACC, D), _f32)

    degp = _deg_call(dst2d, onesD, zerosD)
    h1 = _mm_call(x, W1)
    g1, dis8 = _gscale_call(degp, h1)
    part1 = _agg_call(g1, src2d, dst2d, zerosD)
    g2 = _layer_call(dis8, part1, g1, b1[None, :], W2)
    part2 = _agg_call(g2, src2d, dst2d, zerosD)
    res = _final_call(dis8, part2, g2, b2[None, :], Wlin, blin[None, :])
    return res[0]
